# baseline (speedup 1.0000x reference)
import jax
import jax.numpy as jnp
from jax import lax
from jax.experimental import pallas as pl
from jax.experimental.pallas import tpu as pltpu
from jax.experimental.pallas import tpu_sc as plsc

_N = 32768
_K = 512
_F = 64
_CH = 1024
_GRID = _N // _CH
_NC = 2
_NS = 16
_NW = _NC * _NS
_L = 16
_HCH = 512
_W = 80


def _assign_kernel(x_ref, acc_ref, lab_ref, cen_ref):
    @pl.when(pl.program_id(0) == 0)
    def _():
        a = jnp.sum(acc_ref[...], axis=0)
        cen_ref[...] = a[:, :_F] / a[:, _F:_F + 1]

    c = cen_ref[...]
    x = x_ref[...]
    a2 = jnp.sum(x * x, axis=1, keepdims=True)
    b2 = jnp.sum(c * c, axis=1)
    prod = jax.lax.dot_general(
        x, c, (((1,), (1,)), ((), ())),
        preferred_element_type=jnp.float32,
        precision=jax.lax.Precision.DEFAULT,
    )
    d2 = a2 + b2[None, :] - 2.0 * prod
    d = jnp.sqrt(jnp.maximum(d2, 0.0))
    key = jnp.where(jnp.isnan(d), -jnp.inf, d)
    m = jnp.min(key, axis=1, keepdims=True)
    cols = jax.lax.broadcasted_iota(jnp.int32, key.shape, 1)
    lab_ref[...] = jnp.min(jnp.where(key == m, cols, _K), axis=1)[:, None]


_assign_call = pl.pallas_call(
    _assign_kernel,
    grid=(_GRID,),
    in_specs=[
        pl.BlockSpec((_CH, _F), lambda i: (i, 0)),
        pl.BlockSpec((_NW, _K, _W), lambda i: (0, 0, 0)),
    ],
    out_specs=pl.BlockSpec((_CH, 1), lambda i: (i, 0)),
    out_shape=jax.ShapeDtypeStruct((_N, 1), jnp.int32),
    scratch_shapes=[pltpu.VMEM((_K, _F), jnp.float32)],
    compiler_params=pltpu.CompilerParams(
        dimension_semantics=("arbitrary",)),
)


def _segsum_body(x_hbm, lab_hbm, out_hbm, xv, ivb, acc):
    cid = lax.axis_index("c")
    sid = lax.axis_index("s")
    wid = sid * _NC + cid
    iota = lax.iota(jnp.int32, _L)
    zero16 = jnp.zeros((_L,), jnp.float32)
    e0 = jnp.where(iota == 0, 1.0, 0.0).astype(jnp.float32)

    def _z(i, carry):
        acc[pl.ds(i * _L, _L)] = zero16
        return carry
    lax.fori_loop(0, _K * _W // _L, _z, 0)

    pltpu.sync_copy(lab_hbm.at[wid], ivb)

    for h in range(_CH // _HCH):
        pltpu.sync_copy(x_hbm.at[wid, pl.ds(h * _HCH, _HCH)], xv)

        def _grp(g, carry, h=h):
            labv = ivb[pl.ds(h * _HCH + g * _L, _L)]
            for i in range(_L):
                base16 = jnp.full((_L,), labv[i] * _W, jnp.int32) + iota
                for j in range(_F // _L):
                    v = xv[g * _L + i, pl.ds(j * _L, _L)]
                    plsc.addupdate_scatter(acc, [base16 + j * _L], v)
                plsc.addupdate_scatter(acc, [base16 + _F], e0)
            return carry
        lax.fori_loop(0, _HCH // _L, _grp, 0)

    pltpu.sync_copy(acc, out_hbm.at[wid])


_segsum_call = pl.kernel(
    _segsum_body,
    out_type=jax.ShapeDtypeStruct((_NW, _K * _W), jnp.float32),
    mesh=plsc.VectorSubcoreMesh(core_axis_name="c", subcore_axis_name="s",
                                num_cores=_NC, num_subcores=_NS),
    compiler_params=pltpu.CompilerParams(needs_layout_passes=False),
    scratch_types=[
        pltpu.VMEM((_HCH, _F), jnp.float32),
        pltpu.VMEM((_CH,), jnp.int32),
        pltpu.VMEM((_K * _W,), jnp.float32),
    ],
)


def kernel(X):
    x3 = X.reshape(_NW, _CH, _F)
    init = jnp.zeros((_K, _W), jnp.float32)
    init = init.at[:, :_F].set(X[:_K, :]).at[:, _F].set(1.0)
    acc = jnp.zeros((_NW, _K, _W), jnp.float32).at[0].set(init)
    labels = None
    for it in range(4):
        labels = _assign_call(X, acc)
        if it < 3:
            lab2 = labels.reshape(_NW, _CH)
            acc = _segsum_call(x3, lab2).reshape(_NW, _K, _W)
    return labels.reshape(_N)


# speedup vs baseline: 1.0887x; 1.0887x over previous
import jax
import jax.numpy as jnp
from jax import lax
from jax.experimental import pallas as pl
from jax.experimental.pallas import tpu as pltpu
from jax.experimental.pallas import tpu_sc as plsc

_N = 32768
_K = 512
_F = 64
_CH = 1024
_GRID = _N // _CH
_NC = 2
_NS = 16
_NW = _NC * _NS
_L = 16
_HCH = 512
_W = 80


def _assign_kernel(x_ref, acc_ref, lab_ref, cen_ref, b2_ref):
    @pl.when(pl.program_id(0) == 0)
    def _():
        a = jnp.sum(acc_ref[...], axis=0)
        cnt = a[:, _F:_F + 1]
        cen = a[:, :_F] / cnt
        bad = cnt == 0.0
        cen_ref[...] = jnp.where(bad, 0.0, cen)
        b2 = jnp.sum(cen_ref[...] * cen_ref[...], axis=1)
        b2_ref[...] = jnp.where(bad[:, 0], -1e30, b2)[None, :]

    x = x_ref[...]
    a2 = jnp.sum(x * x, axis=1, keepdims=True)
    prod = jax.lax.dot_general(
        x, cen_ref[...], (((1,), (1,)), ((), ())),
        preferred_element_type=jnp.float32,
        precision=jax.lax.Precision.DEFAULT,
    )
    d2 = a2 + b2_ref[...] - 2.0 * prod
    d = jnp.sqrt(jnp.maximum(d2, 0.0))
    m = jnp.min(d, axis=1, keepdims=True)
    cols = jax.lax.broadcasted_iota(jnp.int32, d.shape, 1)
    lab_ref[...] = jnp.min(jnp.where(d == m, cols, _K), axis=1)[:, None]


_assign_call = pl.pallas_call(
    _assign_kernel,
    grid=(_GRID,),
    in_specs=[
        pl.BlockSpec((_CH, _F), lambda i: (i, 0)),
        pl.BlockSpec((_NW, _K, _W), lambda i: (0, 0, 0)),
    ],
    out_specs=pl.BlockSpec((_CH, 1), lambda i: (i, 0)),
    out_shape=jax.ShapeDtypeStruct((_N, 1), jnp.int32),
    scratch_shapes=[pltpu.VMEM((_K, _F), jnp.float32),
                    pltpu.VMEM((1, _K), jnp.float32)],
    compiler_params=pltpu.CompilerParams(
        dimension_semantics=("arbitrary",)),
)


def _segsum_body(x_hbm, lab_hbm, z_hbm, out_hbm, xv, ivb, acc):
    cid = lax.axis_index("c")
    sid = lax.axis_index("s")
    wid = sid * _NC + cid
    iota = lax.iota(jnp.int32, _L)
    e0 = jnp.where(iota == 0, 1.0, 0.0).astype(jnp.float32)

    pltpu.sync_copy(z_hbm, acc)
    pltpu.sync_copy(lab_hbm.at[wid], ivb)

    for h in range(_CH // _HCH):
        pltpu.sync_copy(x_hbm.at[wid, pl.ds(h * _HCH, _HCH)], xv)

        def _grp(g, carry, h=h):
            labv = ivb[pl.ds(h * _HCH + g * _L, _L)]
            for i in range(_L):
                base16 = jnp.full((_L,), labv[i] * _W, jnp.int32) + iota
                for j in range(_F // _L):
                    v = xv[g * _L + i, pl.ds(j * _L, _L)]
                    plsc.addupdate_scatter(acc, [base16 + j * _L], v)
                plsc.addupdate_scatter(acc, [base16 + _F], e0)
            return carry
        lax.fori_loop(0, _HCH // _L, _grp, 0)

    pltpu.sync_copy(acc, out_hbm.at[wid])


_segsum_call = pl.kernel(
    _segsum_body,
    out_type=jax.ShapeDtypeStruct((_NW, _K * _W), jnp.float32),
    mesh=plsc.VectorSubcoreMesh(core_axis_name="c", subcore_axis_name="s",
                                num_cores=_NC, num_subcores=_NS),
    compiler_params=pltpu.CompilerParams(needs_layout_passes=False),
    scratch_types=[
        pltpu.VMEM((_HCH, _F), jnp.float32),
        pltpu.VMEM((_CH,), jnp.int32),
        pltpu.VMEM((_K * _W,), jnp.float32),
    ],
)


def kernel(X):
    x3 = X.reshape(_NW, _CH, _F)
    zeros = jnp.zeros((_K * _W,), jnp.float32)
    init = jnp.zeros((_K, _W), jnp.float32)
    init = init.at[:, :_F].set(X[:_K, :]).at[:, _F].set(1.0)
    acc = jnp.zeros((_NW, _K, _W), jnp.float32).at[0].set(init)
    labels = None
    for it in range(4):
        labels = _assign_call(X, acc)
        if it < 3:
            lab2 = labels.reshape(_NW, _CH)
            acc = _segsum_call(x3, lab2, zeros).reshape(_NW, _K, _W)
    return labels.reshape(_N)


# speedup vs baseline: 1.2189x; 1.1195x over previous
import jax
import jax.numpy as jnp
from jax import lax
from jax.experimental import pallas as pl
from jax.experimental.pallas import tpu as pltpu
from jax.experimental.pallas import tpu_sc as plsc

_N = 32768
_K = 512
_F = 64
_TCH = 2048
_CH = 1024
_GRID = _N // _TCH
_NC = 2
_NS = 16
_NW = _NC * _NS
_L = 16
_HCH = 512
_W = 80


def _assign_kernel(x_ref, acc_ref, lab_ref, cen_ref, b2_ref):
    @pl.when(pl.program_id(0) == 0)
    def _():
        a = jnp.sum(acc_ref[...], axis=0)
        cnt = a[:, _F:_F + 1]
        cen = a[:, :_F] / cnt
        bad = cnt == 0.0
        cen_ref[...] = jnp.where(bad, 0.0, cen)
        b2 = jnp.sum(cen_ref[...] * cen_ref[...], axis=1)
        b2_ref[...] = jnp.where(bad[:, 0], -1e30, b2)[None, :]

    x = x_ref[...]
    a2 = jnp.sum(x * x, axis=1, keepdims=True)
    prod = jax.lax.dot_general(
        x, cen_ref[...], (((1,), (1,)), ((), ())),
        preferred_element_type=jnp.float32,
        precision=jax.lax.Precision.DEFAULT,
    )
    d2 = a2 + b2_ref[...] - 2.0 * prod
    d = jnp.sqrt(jnp.maximum(d2, 0.0))
    m = jnp.min(d, axis=1, keepdims=True)
    cols = jax.lax.broadcasted_iota(jnp.int32, d.shape, 1)
    lab_ref[...] = jnp.min(jnp.where(d == m, cols, _K), axis=1)[:, None]


_assign_call = pl.pallas_call(
    _assign_kernel,
    grid=(_GRID,),
    in_specs=[
        pl.BlockSpec((_TCH, _F), lambda i: (i, 0)),
        pl.BlockSpec((_NW, _K, _W), lambda i: (0, 0, 0)),
    ],
    out_specs=pl.BlockSpec((_TCH, 1), lambda i: (i, 0)),
    out_shape=jax.ShapeDtypeStruct((_N, 1), jnp.int32),
    scratch_shapes=[pltpu.VMEM((_K, _F), jnp.float32),
                    pltpu.VMEM((1, _K), jnp.float32)],
    compiler_params=pltpu.CompilerParams(
        dimension_semantics=("arbitrary",)),
)


def _segsum_body(x_hbm, lab_hbm, z_hbm, out_hbm, xv, ivb, acc):
    cid = lax.axis_index("c")
    sid = lax.axis_index("s")
    wid = sid * _NC + cid
    iota = lax.iota(jnp.int32, _L)
    e0 = jnp.where(iota == 0, 1.0, 0.0).astype(jnp.float32)

    pltpu.sync_copy(z_hbm, acc)
    pltpu.sync_copy(lab_hbm.at[wid], ivb)

    for h in range(_CH // _HCH):
        pltpu.sync_copy(x_hbm.at[wid, pl.ds(h * _HCH, _HCH)], xv)

        def _grp(g, h=h):
            labv = ivb[pl.ds(h * _HCH + g * _L, _L)]
            for i in range(_L):
                base16 = jnp.full((_L,), labv[i] * _W, jnp.int32) + iota
                for j in range(_F // _L):
                    v = xv[g * _L + i, pl.ds(j * _L, _L)]
                    plsc.addupdate_scatter(acc, [base16 + j * _L], v)
                plsc.addupdate_scatter(acc, [base16 + _F], e0)
        plsc.parallel_loop(0, _HCH // _L, unroll=2)(_grp)

    pltpu.sync_copy(acc, out_hbm.at[wid])


_segsum_call = pl.kernel(
    _segsum_body,
    out_type=jax.ShapeDtypeStruct((_NW, _K * _W), jnp.float32),
    mesh=plsc.VectorSubcoreMesh(core_axis_name="c", subcore_axis_name="s",
                                num_cores=_NC, num_subcores=_NS),
    compiler_params=pltpu.CompilerParams(needs_layout_passes=False),
    scratch_types=[
        pltpu.VMEM((_HCH, _F), jnp.float32),
        pltpu.VMEM((_CH,), jnp.int32),
        pltpu.VMEM((_K * _W,), jnp.float32),
    ],
)


def kernel(X):
    x3 = X.reshape(_NW, _CH, _F)
    zeros = jnp.zeros((_K * _W,), jnp.float32)
    init = jnp.zeros((_K, _W), jnp.float32)
    init = init.at[:, :_F].set(X[:_K, :]).at[:, _F].set(1.0)
    acc = jnp.zeros((_NW, _K, _W), jnp.float32).at[0].set(init)
    labels = None
    for it in range(4):
        labels = _assign_call(X, acc)
        if it < 3:
            lab2 = labels.reshape(_NW, _CH)
            acc = _segsum_call(x3, lab2, zeros).reshape(_NW, _K, _W)
    return labels.reshape(_N)


# speedup vs baseline: 1.2413x; 1.0184x over previous
import jax
import jax.numpy as jnp
from jax import lax
from jax.experimental import pallas as pl
from jax.experimental.pallas import tpu as pltpu
from jax.experimental.pallas import tpu_sc as plsc

_N = 32768
_K = 512
_F = 64
_TCH = 4096
_CH = 1024
_GRID = _N // _TCH
_NC = 2
_NS = 16
_NW = _NC * _NS
_L = 16
_HCH = 512
_W = 80


def _assign_kernel(x_ref, acc_ref, lab_ref, cen_ref, b2_ref):
    @pl.when(pl.program_id(0) == 0)
    def _():
        a = jnp.sum(acc_ref[...], axis=0)
        cnt = a[:, _F:_F + 1]
        cen = a[:, :_F] / cnt
        bad = cnt == 0.0
        cen_ref[...] = jnp.where(bad, 0.0, cen)
        b2 = jnp.sum(cen_ref[...] * cen_ref[...], axis=1)
        b2_ref[...] = jnp.where(bad[:, 0], -1e30, b2)[None, :]

    x = x_ref[...]
    a2 = jnp.sum(x * x, axis=1, keepdims=True)
    prod = jax.lax.dot_general(
        x, cen_ref[...], (((1,), (1,)), ((), ())),
        preferred_element_type=jnp.float32,
        precision=jax.lax.Precision.DEFAULT,
    )
    d2 = a2 + b2_ref[...] - 2.0 * prod
    d = jnp.sqrt(jnp.maximum(d2, 0.0))
    m = jnp.min(d, axis=1, keepdims=True)
    cols = jax.lax.broadcasted_iota(jnp.int32, d.shape, 1)
    lab_ref[...] = jnp.min(jnp.where(d == m, cols, _K), axis=1)[:, None]


_assign_call = pl.pallas_call(
    _assign_kernel,
    grid=(_GRID,),
    in_specs=[
        pl.BlockSpec((_TCH, _F), lambda i: (i, 0)),
        pl.BlockSpec((_NW, _K, _W), lambda i: (0, 0, 0)),
    ],
    out_specs=pl.BlockSpec((_TCH, 1), lambda i: (i, 0)),
    out_shape=jax.ShapeDtypeStruct((_N, 1), jnp.int32),
    scratch_shapes=[pltpu.VMEM((_K, _F), jnp.float32),
                    pltpu.VMEM((1, _K), jnp.float32)],
    compiler_params=pltpu.CompilerParams(
        dimension_semantics=("arbitrary",)),
)


def _segsum_body(x_hbm, lab_hbm, z_hbm, out_hbm, xv, ivb, acc):
    cid = lax.axis_index("c")
    sid = lax.axis_index("s")
    wid = sid * _NC + cid
    iota = lax.iota(jnp.int32, _L)
    e0 = jnp.where(iota == 0, 1.0, 0.0).astype(jnp.float32)

    pltpu.sync_copy(z_hbm, acc)
    pltpu.sync_copy(lab_hbm.at[wid], ivb)

    for h in range(_CH // _HCH):
        pltpu.sync_copy(x_hbm.at[wid, pl.ds(h * _HCH, _HCH)], xv)

        def _grp(g, h=h):
            labv = ivb[pl.ds(h * _HCH + g * _L, _L)]
            for i in range(_L):
                base16 = jnp.full((_L,), labv[i] * _W, jnp.int32) + iota
                for j in range(_F // _L):
                    v = xv[g * _L + i, pl.ds(j * _L, _L)]
                    plsc.addupdate_scatter(acc, [base16 + j * _L], v)
                plsc.addupdate_scatter(acc, [base16 + _F], e0)
        plsc.parallel_loop(0, _HCH // _L, unroll=4)(_grp)

    pltpu.sync_copy(acc, out_hbm.at[wid])


_segsum_call = pl.kernel(
    _segsum_body,
    out_type=jax.ShapeDtypeStruct((_NW, _K * _W), jnp.float32),
    mesh=plsc.VectorSubcoreMesh(core_axis_name="c", subcore_axis_name="s",
                                num_cores=_NC, num_subcores=_NS),
    compiler_params=pltpu.CompilerParams(needs_layout_passes=False),
    scratch_types=[
        pltpu.VMEM((_HCH, _F), jnp.float32),
        pltpu.VMEM((_CH,), jnp.int32),
        pltpu.VMEM((_K * _W,), jnp.float32),
    ],
)


def kernel(X):
    x3 = X.reshape(_NW, _CH, _F)
    zeros = jnp.zeros((_K * _W,), jnp.float32)
    init = jnp.zeros((_K, _W), jnp.float32)
    init = init.at[:, :_F].set(X[:_K, :]).at[:, _F].set(1.0)
    acc = jnp.zeros((_NW, _K, _W), jnp.float32).at[0].set(init)
    labels = None
    for it in range(4):
        labels = _assign_call(X, acc)
        if it < 3:
            lab2 = labels.reshape(_NW, _CH)
            acc = _segsum_call(x3, lab2, zeros).reshape(_NW, _K, _W)
    return labels.reshape(_N)
